# baseline (speedup 1.0000x reference)
import jax
import jax.numpy as jnp
from jax.experimental import pallas as pl
from jax.experimental.pallas import tpu as pltpu

_DJ = 8


def _fire_kernel(x_ref, pos_ref, tab_ref, out_ref, lhs_ref, pos8_ref):
    dj = pl.program_id(0)
    B, P = x_ref.shape
    L = tab_ref.shape[0]
    Dt = tab_ref.shape[1]

    pos8_ref[...] = pos_ref[...].astype(jnp.float8_e4m3fn)
    pos8 = pos8_ref[...]
    tab = tab_ref[...]
    for i in range(B):
        @pl.when(dj == 0)
        def _build_lhs(i=i):
            row = x_ref[i:i + 1, :]
            idx = (row * float(L - 1)).astype(jnp.int32)
            lv = jax.lax.broadcasted_iota(jnp.int32, (L, P), 0)
            lhs_ref[i * L:(i + 1) * L, :] = (lv == idx).astype(
                jnp.float8_e4m3fn)

        qb = jnp.dot(lhs_ref[i * L:(i + 1) * L, :], pos8,
                     preferred_element_type=jnp.float32)
        msb = jnp.sum(qb * tab, axis=0)
        out_ref[i, :] = jnp.where(msb > 0, 1.0, -1.0).astype(jnp.float32)


def kernel(x, position, value_table):
    B = x.shape[0]
    flat = x.reshape(B, -1)
    P = flat.shape[1]
    L, D = value_table.shape
    Dt = D // _DJ
    return pl.pallas_call(
        _fire_kernel,
        grid=(_DJ,),
        in_specs=[
            pl.BlockSpec((B, P), lambda dj: (0, 0)),
            pl.BlockSpec((P, Dt), lambda dj: (0, dj)),
            pl.BlockSpec((L, Dt), lambda dj: (0, dj)),
        ],
        out_specs=pl.BlockSpec((B, Dt), lambda dj: (0, dj)),
        out_shape=jax.ShapeDtypeStruct((B, D), jnp.float32),
        scratch_shapes=[
            pltpu.VMEM((B * L, P), jnp.float8_e4m3fn),
            pltpu.VMEM((P, Dt), jnp.float8_e4m3fn),
        ],
    )(flat, position, value_table)


# speedup vs baseline: 1.2991x; 1.2991x over previous
import jax
import jax.numpy as jnp
from jax.experimental import pallas as pl
from jax.experimental.pallas import tpu as pltpu

_DJ = 8


def _fire_kernel(x_ref, pos_ref, tab_ref, out_ref, lhs_ref, pos8_ref):
    dj = pl.program_id(0)
    B, P = x_ref.shape
    L = tab_ref.shape[0]
    Dt = tab_ref.shape[1]

    @pl.when(dj == 0)
    def _build_lhs():
        def body(i, carry):
            row = x_ref[pl.ds(i, 1), :]
            idx = (row * float(L - 1)).astype(jnp.int32)
            lv = jax.lax.broadcasted_iota(jnp.int32, (L, P), 0)
            lhs_ref[pl.ds(i * L, L), :] = (lv == idx).astype(jnp.float8_e4m3fn)
            return carry
        jax.lax.fori_loop(0, B, body, 0)

    pos8_ref[...] = pos_ref[...].astype(jnp.float8_e4m3fn)
    pos8 = pos8_ref[...]
    tab = tab_ref[...]
    for i in range(B):
        qb = jnp.dot(lhs_ref[i * L:(i + 1) * L, :], pos8,
                     preferred_element_type=jnp.float32)
        msb = jnp.sum(qb * tab, axis=0)
        out_ref[i, :] = jnp.where(msb > 0, 1.0, -1.0).astype(jnp.float32)


def kernel(x, position, value_table):
    B = x.shape[0]
    flat = x.reshape(B, -1)
    P = flat.shape[1]
    L, D = value_table.shape
    Dt = D // _DJ
    return pl.pallas_call(
        _fire_kernel,
        grid=(_DJ,),
        in_specs=[
            pl.BlockSpec((B, P), lambda dj: (0, 0)),
            pl.BlockSpec((P, Dt), lambda dj: (0, dj)),
            pl.BlockSpec((L, Dt), lambda dj: (0, dj)),
        ],
        out_specs=pl.BlockSpec((B, Dt), lambda dj: (0, dj)),
        out_shape=jax.ShapeDtypeStruct((B, D), jnp.float32),
        scratch_shapes=[
            pltpu.VMEM((B * L, P), jnp.float8_e4m3fn),
            pltpu.VMEM((P, Dt), jnp.float8_e4m3fn),
        ],
    )(flat, position, value_table)


# speedup vs baseline: 1.3270x; 1.0215x over previous
import jax
import jax.numpy as jnp
from jax.experimental import pallas as pl
from jax.experimental.pallas import tpu as pltpu

_DJ = 8


def _fire_kernel(x_ref, pos_ref, tab_ref, out_ref, lhs_ref, pos8_ref):
    dj = pl.program_id(0)
    B = x_ref.shape[0]
    P = x_ref.size // B
    L = tab_ref.shape[0]
    Dt = tab_ref.shape[1]

    @pl.when(dj == 0)
    def _build_lhs():
        flat = x_ref[...].reshape(B, P)
        idx_all = (flat * float(L - 1)).astype(jnp.int32)
        lv = jax.lax.broadcasted_iota(jnp.int32, (L, P), 0)
        for i in range(B):
            lhs_ref[i * L:(i + 1) * L, :] = (
                lv == idx_all[i:i + 1, :]).astype(jnp.float8_e4m3fn)

    pos8_ref[...] = pos_ref[...].astype(jnp.float8_e4m3fn)
    pos8 = pos8_ref[...]
    tab = tab_ref[...]
    for i in range(B):
        qb = jnp.dot(lhs_ref[i * L:(i + 1) * L, :], pos8,
                     preferred_element_type=jnp.float32)
        msb = jnp.sum(qb * tab, axis=0)
        out_ref[i, :] = jnp.where(msb > 0, 1.0, -1.0).astype(jnp.float32)


def kernel(x, position, value_table):
    B = x.shape[0]
    P = x.size // B
    L, D = value_table.shape
    Dt = D // _DJ
    return pl.pallas_call(
        _fire_kernel,
        grid=(_DJ,),
        in_specs=[
            pl.BlockSpec(x.shape, lambda dj: (0,) * x.ndim),
            pl.BlockSpec((P, Dt), lambda dj: (0, dj)),
            pl.BlockSpec((L, Dt), lambda dj: (0, dj)),
        ],
        out_specs=pl.BlockSpec((B, Dt), lambda dj: (0, dj)),
        out_shape=jax.ShapeDtypeStruct((B, D), jnp.float32),
        scratch_shapes=[
            pltpu.VMEM((B * L, P), jnp.float8_e4m3fn),
            pltpu.VMEM((P, Dt), jnp.float8_e4m3fn),
        ],
    )(x, position, value_table)
